# baseline (speedup 1.0000x reference)
import functools

import jax
import jax.numpy as jnp
from jax import lax
from jax.experimental import pallas as pl
from jax.experimental.pallas import tpu as pltpu
from jax.experimental.pallas import tpu_sc as plsc

N = 10000
E = 320000
D = 128

NC, NS, LANES = 2, 16, 16
NW = NC * NS
NPAD = 10240
RPT = NPAD // NS
CH = 128
ZCH = RPT // CH
BLK = 8

ET = E + N
CPT = 88
NB = CPT // BLK
TPT = CPT * CH
EPAD = NW * TPT

assert CPT * NW * CH >= ET

_SC_PARAMS = dict(
    mesh=plsc.VectorSubcoreMesh(core_axis_name="c", subcore_axis_name="s"),
    compiler_params=pltpu.CompilerParams(needs_layout_passes=False),
)


@functools.cache
def _get_deg_kernel():
    return pl.kernel(
        _deg_body,
        out_type=jax.ShapeDtypeStruct((NC * NPAD,), jnp.float32),
        mesh=plsc.VectorSubcoreMesh(core_axis_name="c", subcore_axis_name="s"),
        compiler_params=pltpu.CompilerParams(needs_layout_passes=False),
        scratch_types=[
            pltpu.VMEM((CPT, CH), jnp.int32),
            pltpu.VMEM((CH,), jnp.float32),
            pltpu.VMEM((RPT,), jnp.float32),
            pltpu.VMEM_SHARED((NPAD,), jnp.float32),
            pltpu.SemaphoreType.DMA,
        ],
    )


def _deg_body(rows_hbm, out_hbm, idx_v, val_v, zbuf, deg_sh, sem):
    del sem
    cid = lax.axis_index("c")
    sid = lax.axis_index("s")
    wid = cid * NS + sid
    for k in range(RPT // LANES):
        zbuf[pl.ds(k * LANES, LANES)] = jnp.zeros((LANES,), jnp.float32)
    pltpu.sync_copy(zbuf, deg_sh.at[pl.ds(sid * RPT, RPT)])
    pltpu.sync_copy(rows_hbm.at[wid], idx_v)
    plsc.subcore_barrier()
    base = wid * TPT

    def chunk(j, carry):
        for k in range(CH // LANES):
            eid = base + j * CH + k * LANES + lax.iota(jnp.int32, LANES)
            val_v[pl.ds(k * LANES, LANES)] = jnp.where(
                eid < E, jnp.float32(1.0), jnp.float32(0.0))
        pltpu.sync_copy(val_v, deg_sh.at[idx_v.at[j]], add=True)
        return carry

    lax.fori_loop(0, CPT, chunk, 0)
    plsc.subcore_barrier()
    pltpu.sync_copy(deg_sh.at[pl.ds(sid * RPT, RPT)], zbuf)
    pltpu.sync_copy(zbuf, out_hbm.at[pl.ds(cid * NPAD + sid * RPT, RPT)])


@functools.cache
def _get_edge_kernel():
    return pl.kernel(
        _edge_body,
        out_type=jax.ShapeDtypeStruct((NW, CPT, CH), jnp.float32),
        mesh=plsc.VectorSubcoreMesh(core_axis_name="c", subcore_axis_name="s"),
        compiler_params=pltpu.CompilerParams(needs_layout_passes=False),
        scratch_types=[
            pltpu.VMEM((NPAD,), jnp.float32),
            pltpu.VMEM((NPAD,), jnp.float32),
            pltpu.VMEM((NPAD,), jnp.float32),
            pltpu.VMEM((2, BLK, CH), jnp.int32),
            pltpu.VMEM((2, BLK, CH), jnp.int32),
            pltpu.VMEM((2, BLK, CH), jnp.float32),
            pltpu.SemaphoreType.DMA,
            pltpu.SemaphoreType.DMA,
        ],
    )


def _edge_body(row_hbm, col_hbm, p_hbm, q_hbm, l_hbm,
               w_hbm, p_v, q_v, l_v, ridx, cidx, w_v, sem_i, sem_o):
    cid = lax.axis_index("c")
    sid = lax.axis_index("s")
    wid = cid * NS + sid
    pltpu.sync_copy(p_hbm, p_v)
    pltpu.sync_copy(q_hbm, q_v)
    pltpu.sync_copy(l_hbm, l_v)
    pltpu.sync_copy(row_hbm.at[wid, pl.ds(0, BLK)], ridx.at[0])
    pltpu.sync_copy(col_hbm.at[wid, pl.ds(0, BLK)], cidx.at[0])

    def block(b, carry):
        cur = lax.rem(b, 2)
        nxt = 1 - cur

        @pl.when(b + 1 < NB)
        def _prefetch():
            pltpu.async_copy(row_hbm.at[wid, pl.ds((b + 1) * BLK, BLK)],
                             ridx.at[nxt], sem_i)
            pltpu.async_copy(col_hbm.at[wid, pl.ds((b + 1) * BLK, BLK)],
                             cidx.at[nxt], sem_i)

        @pl.when(b >= 2)
        def _drain_write():
            pltpu.make_async_copy(
                w_v.at[0], w_hbm.at[wid, pl.ds(0, BLK)], sem_o).wait()

        @plsc.parallel_loop(0, BLK, unroll=2)
        def chunk(m):
            base = wid * TPT + (b * BLK + m) * CH
            for k in range(CH // LANES):
                sl = pl.ds(k * LANES, LANES)
                rv = ridx[cur, m, sl]
                cv = cidx[cur, m, sl]
                pr = plsc.load_gather(p_v, [rv])
                qc = plsc.load_gather(q_v, [cv])
                lr = plsc.load_gather(l_v, [rv])
                lc = plsc.load_gather(l_v, [cv])
                eid = base + k * LANES + lax.iota(jnp.int32, LANES)
                w = jnp.exp(-(pr + qc) * (lr + lc))
                w_v[cur, m, sl] = jnp.where(eid < ET, w, jnp.float32(0.0))

        pltpu.async_copy(w_v.at[cur], w_hbm.at[wid, pl.ds(b * BLK, BLK)],
                         sem_o)

        @pl.when(b + 1 < NB)
        def _wait_prefetch():
            for _ in range(2):
                pltpu.make_async_copy(
                    row_hbm.at[wid, pl.ds(0, BLK)], ridx.at[0], sem_i).wait()

        return carry

    lax.fori_loop(0, NB, block, 0)
    for _ in range(2):
        pltpu.make_async_copy(
            w_v.at[0], w_hbm.at[wid, pl.ds(0, BLK)], sem_o).wait()


@functools.cache
def _get_agg_kernel():
    return pl.kernel(
        _agg_body,
        out_type=jax.ShapeDtypeStruct((NC, NPAD, D), jnp.float32),
        mesh=plsc.VectorSubcoreMesh(core_axis_name="c", subcore_axis_name="s"),
        compiler_params=pltpu.CompilerParams(needs_layout_passes=False),
        scratch_types=[
            pltpu.VMEM((2, BLK, CH), jnp.int32),
            pltpu.VMEM((2, BLK, CH), jnp.int32),
            pltpu.VMEM((2, BLK, CH), jnp.float32),
            pltpu.VMEM((2, CH, D), jnp.float32),
            pltpu.VMEM_SHARED((NPAD, D), jnp.float32),
            pltpu.SemaphoreType.DMA,
            pltpu.SemaphoreType.DMA,
            pltpu.SemaphoreType.DMA,
        ],
    )


def _agg_body(row_hbm, col_hbm, ps_hbm, w_hbm,
              out_hbm, ridx, cidx, w_v, rows_v, acc_sh, sem_g, sem_s, sem_i):
    cid = lax.axis_index("c")
    sid = lax.axis_index("s")
    wid = cid * NS + sid

    def _wait_gather(bb):
        pltpu.make_async_copy(
            ps_hbm.at[cidx.at[0, 0]], rows_v.at[bb], sem_g).wait()

    def _wait_scatter():
        pltpu.make_async_copy(
            rows_v.at[0], acc_sh.at[ridx.at[0, 0]], sem_s).wait()

    def _wait_idx():
        pltpu.make_async_copy(
            row_hbm.at[wid, pl.ds(0, BLK)], ridx.at[0], sem_i).wait()

    def zrow(r, c0):
        for k in range(D // LANES):
            rows_v[0, r, pl.ds(k * LANES, LANES)] = jnp.zeros((LANES,),
                                                              jnp.float32)
        return c0

    lax.fori_loop(0, CH, zrow, 0)
    for t in range(ZCH):
        pltpu.sync_copy(rows_v.at[0], acc_sh.at[pl.ds(sid * RPT + t * CH, CH)])
    plsc.subcore_barrier()

    pltpu.sync_copy(row_hbm.at[wid, pl.ds(0, BLK)], ridx.at[0])
    pltpu.sync_copy(col_hbm.at[wid, pl.ds(0, BLK)], cidx.at[0])
    pltpu.sync_copy(w_hbm.at[wid, pl.ds(0, BLK)], w_v.at[0])
    pltpu.async_copy(ps_hbm.at[cidx.at[0, 0]], rows_v.at[0], sem_g)

    def block(b, carry):
        cur = lax.rem(b, 2)
        nxt = 1 - cur

        @pl.when(b + 1 < NB)
        def _prefetch():
            pltpu.async_copy(row_hbm.at[wid, pl.ds((b + 1) * BLK, BLK)],
                             ridx.at[nxt], sem_i)
            pltpu.async_copy(col_hbm.at[wid, pl.ds((b + 1) * BLK, BLK)],
                             cidx.at[nxt], sem_i)
            pltpu.async_copy(w_hbm.at[wid, pl.ds((b + 1) * BLK, BLK)],
                             w_v.at[nxt], sem_i)

        for m in range(BLK):
            bb = m % 2
            _wait_gather(bb)
            if m + 1 < BLK:
                if m >= 1:
                    _wait_scatter()
                pltpu.async_copy(ps_hbm.at[cidx.at[cur, m + 1]],
                                 rows_v.at[(m + 1) % 2], sem_g)
            else:
                _wait_scatter()

                @pl.when(b + 1 < NB)
                def _cross():
                    for _ in range(3):
                        _wait_idx()
                    pltpu.async_copy(ps_hbm.at[cidx.at[nxt, 0]],
                                     rows_v.at[0], sem_g)

            @plsc.parallel_loop(0, CH // LANES, unroll=2)
            def scale(g, _m=m, _bb=bb):
                wv = w_v[cur, _m, pl.ds(g * LANES, LANES)]
                for i in range(LANES):
                    ws = lax.gather(
                        wv, jnp.broadcast_to(i, (LANES, 1)),
                        lax.GatherDimensionNumbers(
                            offset_dims=(), collapsed_slice_dims=(0,),
                            start_index_map=(0,)),
                        slice_sizes=(1,),
                        mode=lax.GatherScatterMode.PROMISE_IN_BOUNDS)
                    e = g * LANES + i
                    for k in range(D // LANES):
                        sl = pl.ds(k * LANES, LANES)
                        rows_v[_bb, e, sl] = rows_v[_bb, e, sl] * ws

            pltpu.async_copy(rows_v.at[bb], acc_sh.at[ridx.at[cur, m]],
                             sem_s, add=True)
        _wait_scatter()
        return carry

    lax.fori_loop(0, NB, block, 0)
    plsc.subcore_barrier()
    for t in range(ZCH):
        pltpu.sync_copy(acc_sh.at[pl.ds(sid * RPT + t * CH, CH)],
                        out_hbm.at[cid, pl.ds(sid * RPT + t * CH, CH)])


def _pre_body(x_ref, w_ref, f1_ref, f2_ref, fb_ref, degp_ref,
              ps_ref, p_ref, q_ref, l_ref):
    x = jnp.concatenate(
        [x_ref[...], jnp.zeros((NPAD - N, D), jnp.float32)], axis=0)
    ps = jnp.dot(x, w_ref[...], preferred_element_type=jnp.float32)
    ps_ref[...] = ps
    a = jnp.dot(ps, f1_ref[...], preferred_element_type=jnp.float32)
    b = jnp.dot(ps, f2_ref[...], preferred_element_type=jnp.float32)
    p_ref[...] = a[:, 0]
    q_ref[...] = b[:, 0] + fb_ref[0]
    deg = degp_ref[0] + degp_ref[1] + 1.0
    l_ref[...] = jnp.log(deg)


_pre_call = pl.pallas_call(
    _pre_body,
    out_shape=(
        jax.ShapeDtypeStruct((NPAD, D), jnp.float32),
        jax.ShapeDtypeStruct((NPAD,), jnp.float32),
        jax.ShapeDtypeStruct((NPAD,), jnp.float32),
        jax.ShapeDtypeStruct((NPAD,), jnp.float32),
    ),
    in_specs=[
        pl.BlockSpec(),
        pl.BlockSpec(),
        pl.BlockSpec(),
        pl.BlockSpec(),
        pl.BlockSpec(memory_space=pltpu.SMEM),
        pl.BlockSpec(),
    ],
)


def _post_body(parts_ref, o_ref):
    s = parts_ref[0, :N, :] + parts_ref[1, :N, :]
    o_ref[...] = jnp.maximum(s, 0.0)


_post_call = pl.pallas_call(
    _post_body,
    out_shape=jax.ShapeDtypeStruct((N, D), jnp.float32),
)


def kernel(x, edge_index, W, f_weights, f_bias):
    diag = jnp.arange(N, dtype=edge_index.dtype)
    row = jnp.concatenate([edge_index[0], diag])
    col = jnp.concatenate([edge_index[1], diag])
    epad_ids = N + jnp.arange(EPAD - ET, dtype=edge_index.dtype) % (NPAD - N)
    row_p = jnp.concatenate([row, epad_ids]).reshape(NW, CPT, CH)
    col_p = jnp.concatenate([col, epad_ids]).reshape(NW, CPT, CH)

    deg_parts = _get_deg_kernel()(row_p).reshape(NC, NPAD)
    ps, P, Q, Lg = _pre_call(x, W, f_weights[:D], f_weights[D:], f_bias,
                             deg_parts)
    w = _get_edge_kernel()(row_p, col_p, P, Q, Lg)
    parts = _get_agg_kernel()(row_p, col_p, ps, w)
    return _post_call(parts)


# speedup vs baseline: 1.0840x; 1.0840x over previous
import functools

import jax
import jax.numpy as jnp
from jax import lax
from jax.experimental import pallas as pl
from jax.experimental.pallas import tpu as pltpu
from jax.experimental.pallas import tpu_sc as plsc

N = 10000
E = 320000
D = 128

NC, NS, LANES = 2, 16, 16
NW = NC * NS
NPAD = 10240
RPT = NPAD // NS
CH = 128
ZCH = RPT // CH
BLK = 8

ET = E + N
CPT = 88
NB = CPT // BLK
TPT = CPT * CH
EPAD = NW * TPT

assert CPT * NW * CH >= ET

_SC_PARAMS = dict(
    mesh=plsc.VectorSubcoreMesh(core_axis_name="c", subcore_axis_name="s"),
    compiler_params=pltpu.CompilerParams(needs_layout_passes=False),
)


@functools.cache
def _get_deg_kernel():
    return pl.kernel(
        _deg_body,
        out_type=jax.ShapeDtypeStruct((NC * NPAD,), jnp.float32),
        mesh=plsc.VectorSubcoreMesh(core_axis_name="c", subcore_axis_name="s"),
        compiler_params=pltpu.CompilerParams(needs_layout_passes=False),
        scratch_types=[
            pltpu.VMEM((CPT, CH), jnp.int32),
            pltpu.VMEM((CH,), jnp.float32),
            pltpu.VMEM((RPT,), jnp.float32),
            pltpu.VMEM_SHARED((NPAD,), jnp.float32),
            pltpu.SemaphoreType.DMA,
        ],
    )


def _deg_body(rows_hbm, out_hbm, idx_v, val_v, zbuf, deg_sh, sem):
    del sem
    cid = lax.axis_index("c")
    sid = lax.axis_index("s")
    wid = cid * NS + sid
    for k in range(RPT // LANES):
        zbuf[pl.ds(k * LANES, LANES)] = jnp.zeros((LANES,), jnp.float32)
    pltpu.sync_copy(zbuf, deg_sh.at[pl.ds(sid * RPT, RPT)])
    pltpu.sync_copy(rows_hbm.at[wid], idx_v)
    plsc.subcore_barrier()
    base = wid * TPT

    def chunk(j, carry):
        for k in range(CH // LANES):
            eid = base + j * CH + k * LANES + lax.iota(jnp.int32, LANES)
            val_v[pl.ds(k * LANES, LANES)] = jnp.where(
                eid < E, jnp.float32(1.0), jnp.float32(0.0))
        pltpu.sync_copy(val_v, deg_sh.at[idx_v.at[j]], add=True)
        return carry

    lax.fori_loop(0, CPT, chunk, 0)
    plsc.subcore_barrier()
    pltpu.sync_copy(deg_sh.at[pl.ds(sid * RPT, RPT)], zbuf)
    pltpu.sync_copy(zbuf, out_hbm.at[pl.ds(cid * NPAD + sid * RPT, RPT)])


@functools.cache
def _get_edge_kernel():
    return pl.kernel(
        _edge_body,
        out_type=jax.ShapeDtypeStruct((NW, CPT, CH), jnp.float32),
        mesh=plsc.VectorSubcoreMesh(core_axis_name="c", subcore_axis_name="s"),
        compiler_params=pltpu.CompilerParams(needs_layout_passes=False),
        scratch_types=[
            pltpu.VMEM((NPAD,), jnp.float32),
            pltpu.VMEM((NPAD,), jnp.float32),
            pltpu.VMEM((NPAD,), jnp.float32),
            pltpu.VMEM((2, BLK, CH), jnp.int32),
            pltpu.VMEM((2, BLK, CH), jnp.int32),
            pltpu.VMEM((2, BLK, CH), jnp.float32),
            pltpu.SemaphoreType.DMA,
            pltpu.SemaphoreType.DMA,
        ],
    )


def _edge_body(row_hbm, col_hbm, p_hbm, q_hbm, l_hbm,
               w_hbm, p_v, q_v, l_v, ridx, cidx, w_v, sem_i, sem_o):
    cid = lax.axis_index("c")
    sid = lax.axis_index("s")
    wid = cid * NS + sid
    pltpu.sync_copy(p_hbm, p_v)
    pltpu.sync_copy(q_hbm, q_v)
    pltpu.sync_copy(l_hbm, l_v)
    pltpu.sync_copy(row_hbm.at[wid, pl.ds(0, BLK)], ridx.at[0])
    pltpu.sync_copy(col_hbm.at[wid, pl.ds(0, BLK)], cidx.at[0])

    def block(b, carry):
        cur = lax.rem(b, 2)
        nxt = 1 - cur

        @pl.when(b + 1 < NB)
        def _prefetch():
            pltpu.async_copy(row_hbm.at[wid, pl.ds((b + 1) * BLK, BLK)],
                             ridx.at[nxt], sem_i)
            pltpu.async_copy(col_hbm.at[wid, pl.ds((b + 1) * BLK, BLK)],
                             cidx.at[nxt], sem_i)

        @pl.when(b >= 2)
        def _drain_write():
            pltpu.make_async_copy(
                w_v.at[0], w_hbm.at[wid, pl.ds(0, BLK)], sem_o).wait()

        @plsc.parallel_loop(0, BLK, unroll=2)
        def chunk(m):
            base = wid * TPT + (b * BLK + m) * CH
            for k in range(CH // LANES):
                sl = pl.ds(k * LANES, LANES)
                rv = ridx[cur, m, sl]
                cv = cidx[cur, m, sl]
                pr = plsc.load_gather(p_v, [rv])
                qc = plsc.load_gather(q_v, [cv])
                lr = plsc.load_gather(l_v, [rv])
                lc = plsc.load_gather(l_v, [cv])
                eid = base + k * LANES + lax.iota(jnp.int32, LANES)
                w = jnp.exp(-(pr + qc) * (lr + lc))
                w_v[cur, m, sl] = jnp.where(eid < ET, w, jnp.float32(0.0))

        pltpu.async_copy(w_v.at[cur], w_hbm.at[wid, pl.ds(b * BLK, BLK)],
                         sem_o)

        @pl.when(b + 1 < NB)
        def _wait_prefetch():
            for _ in range(2):
                pltpu.make_async_copy(
                    row_hbm.at[wid, pl.ds(0, BLK)], ridx.at[0], sem_i).wait()

        return carry

    lax.fori_loop(0, NB, block, 0)
    for _ in range(2):
        pltpu.make_async_copy(
            w_v.at[0], w_hbm.at[wid, pl.ds(0, BLK)], sem_o).wait()


@functools.cache
def _get_agg_kernel():
    return pl.kernel(
        _agg_body,
        out_type=jax.ShapeDtypeStruct((NC, NPAD, D), jnp.float32),
        mesh=plsc.VectorSubcoreMesh(core_axis_name="c", subcore_axis_name="s"),
        compiler_params=pltpu.CompilerParams(needs_layout_passes=False),
        scratch_types=[
            pltpu.VMEM((2, BLK, CH), jnp.int32),
            pltpu.VMEM((2, BLK, CH), jnp.int32),
            pltpu.VMEM((2, BLK, CH), jnp.float32),
            pltpu.VMEM((2, CH, D), jnp.float32),
            pltpu.VMEM_SHARED((NPAD, D), jnp.float32),
            pltpu.SemaphoreType.DMA,
            pltpu.SemaphoreType.DMA,
            pltpu.SemaphoreType.DMA,
        ],
    )


def _agg_body(row_hbm, col_hbm, ps_hbm, w_hbm,
              out_hbm, ridx, cidx, w_v, rows_v, acc_sh, sem_g, sem_s, sem_i):
    cid = lax.axis_index("c")
    sid = lax.axis_index("s")
    wid = cid * NS + sid

    def _wait_gather(bb):
        pltpu.make_async_copy(
            ps_hbm.at[cidx.at[0, 0]], rows_v.at[bb], sem_g).wait()

    def _wait_scatter():
        pltpu.make_async_copy(
            rows_v.at[0], acc_sh.at[ridx.at[0, 0]], sem_s).wait()

    def _wait_idx():
        pltpu.make_async_copy(
            row_hbm.at[wid, pl.ds(0, BLK)], ridx.at[0], sem_i).wait()

    def zrow(r, c0):
        for k in range(D // LANES):
            rows_v[0, r, pl.ds(k * LANES, LANES)] = jnp.zeros((LANES,),
                                                              jnp.float32)
        return c0

    lax.fori_loop(0, CH, zrow, 0)
    for t in range(ZCH):
        pltpu.sync_copy(rows_v.at[0], acc_sh.at[pl.ds(sid * RPT + t * CH, CH)])
    plsc.subcore_barrier()

    pltpu.sync_copy(row_hbm.at[wid, pl.ds(0, BLK)], ridx.at[0])
    pltpu.sync_copy(col_hbm.at[wid, pl.ds(0, BLK)], cidx.at[0])
    pltpu.sync_copy(w_hbm.at[wid, pl.ds(0, BLK)], w_v.at[0])
    pltpu.async_copy(ps_hbm.at[cidx.at[0, 0]], rows_v.at[0], sem_g)

    def block(b, carry):
        cur = lax.rem(b, 2)
        nxt = 1 - cur

        @pl.when(b + 1 < NB)
        def _prefetch():
            pltpu.async_copy(row_hbm.at[wid, pl.ds((b + 1) * BLK, BLK)],
                             ridx.at[nxt], sem_i)
            pltpu.async_copy(col_hbm.at[wid, pl.ds((b + 1) * BLK, BLK)],
                             cidx.at[nxt], sem_i)
            pltpu.async_copy(w_hbm.at[wid, pl.ds((b + 1) * BLK, BLK)],
                             w_v.at[nxt], sem_i)

        for m in range(BLK):
            bb = m % 2
            _wait_gather(bb)
            if m + 1 < BLK:
                if m >= 1:
                    _wait_scatter()
                pltpu.async_copy(ps_hbm.at[cidx.at[cur, m + 1]],
                                 rows_v.at[(m + 1) % 2], sem_g)
            else:
                _wait_scatter()

                @pl.when(b + 1 < NB)
                def _cross():
                    for _ in range(3):
                        _wait_idx()
                    pltpu.async_copy(ps_hbm.at[cidx.at[nxt, 0]],
                                     rows_v.at[0], sem_g)

            @plsc.parallel_loop(0, CH // LANES, unroll=2)
            def scale(g, _m=m, _bb=bb):
                wv = w_v[cur, _m, pl.ds(g * LANES, LANES)]
                for i in range(LANES):
                    ws = lax.gather(
                        wv, jnp.broadcast_to(i, (LANES, 1)),
                        lax.GatherDimensionNumbers(
                            offset_dims=(), collapsed_slice_dims=(0,),
                            start_index_map=(0,)),
                        slice_sizes=(1,),
                        mode=lax.GatherScatterMode.PROMISE_IN_BOUNDS)
                    e = g * LANES + i
                    for k in range(D // LANES):
                        sl = pl.ds(k * LANES, LANES)
                        rows_v[_bb, e, sl] = rows_v[_bb, e, sl] * ws

            pltpu.async_copy(rows_v.at[bb], acc_sh.at[ridx.at[cur, m]],
                             sem_s, add=True)
        _wait_scatter()
        return carry

    lax.fori_loop(0, NB, block, 0)
    plsc.subcore_barrier()
    for t in range(ZCH):
        pltpu.sync_copy(acc_sh.at[pl.ds(sid * RPT + t * CH, CH)],
                        out_hbm.at[cid, pl.ds(sid * RPT + t * CH, CH)])


def _pre_body(x_ref, w_ref, f1_ref, f2_ref, fb_ref, degp_ref,
              ps_ref, p_ref, q_ref, l_ref):
    x = jnp.concatenate(
        [x_ref[...], jnp.zeros((NPAD - N, D), jnp.float32)], axis=0)
    ps = jnp.dot(x, w_ref[...], preferred_element_type=jnp.float32)
    ps_ref[...] = ps
    a = jnp.dot(ps, f1_ref[...], preferred_element_type=jnp.float32)
    b = jnp.dot(ps, f2_ref[...], preferred_element_type=jnp.float32)
    p_ref[...] = a[:, 0]
    q_ref[...] = b[:, 0] + fb_ref[0]
    deg = degp_ref[0] + degp_ref[1] + 1.0
    l_ref[...] = jnp.log(deg)


_pre_call = pl.pallas_call(
    _pre_body,
    out_shape=(
        jax.ShapeDtypeStruct((NPAD, D), jnp.float32),
        jax.ShapeDtypeStruct((NPAD,), jnp.float32),
        jax.ShapeDtypeStruct((NPAD,), jnp.float32),
        jax.ShapeDtypeStruct((NPAD,), jnp.float32),
    ),
    in_specs=[
        pl.BlockSpec(),
        pl.BlockSpec(),
        pl.BlockSpec(),
        pl.BlockSpec(),
        pl.BlockSpec(memory_space=pltpu.SMEM),
        pl.BlockSpec(),
    ],
)


def _prep_body(ei_ref, row_ref, col_ref):
    pos = lax.iota(jnp.int32, EPAD)
    tail = jnp.where(
        pos < ET, pos - E, N + lax.rem(pos - ET, jnp.int32(NPAD - N)))
    zpad = jnp.zeros((EPAD - E,), jnp.int32)
    row = jnp.concatenate([ei_ref[0, :], zpad])
    col = jnp.concatenate([ei_ref[1, :], zpad])
    row_ref[...] = jnp.where(pos < E, row, tail).reshape(NW, CPT, CH)
    col_ref[...] = jnp.where(pos < E, col, tail).reshape(NW, CPT, CH)


_prep_call = pl.pallas_call(
    _prep_body,
    out_shape=(
        jax.ShapeDtypeStruct((NW, CPT, CH), jnp.int32),
        jax.ShapeDtypeStruct((NW, CPT, CH), jnp.int32),
    ),
)


def _post_body(parts_ref, o_ref):
    s = parts_ref[0, :N, :] + parts_ref[1, :N, :]
    o_ref[...] = jnp.maximum(s, 0.0)


_post_call = pl.pallas_call(
    _post_body,
    out_shape=jax.ShapeDtypeStruct((N, D), jnp.float32),
)


def kernel(x, edge_index, W, f_weights, f_bias):
    row_p, col_p = _prep_call(edge_index)

    deg_parts = _get_deg_kernel()(row_p).reshape(NC, NPAD)
    ps, P, Q, Lg = _pre_call(x, W, f_weights[:D], f_weights[D:], f_bias,
                             deg_parts)
    w = _get_edge_kernel()(row_p, col_p, P, Q, Lg)
    parts = _get_agg_kernel()(row_p, col_p, ps, w)
    return _post_call(parts)
